# baseline (speedup 1.0000x reference)
import functools

import jax
import jax.numpy as jnp
from jax import lax
from jax.experimental import pallas as pl
from jax.experimental.pallas import tpu as pltpu
from jax.experimental.pallas import tpu_sc as plsc

_SCALE = 65536.0
_N = 16777216

_NW = 32
_SHARD = _N // _NW
_CHUNK = 16384
_NCHUNK = _SHARD // _CHUNK
_VPC = _CHUNK // 16
_UNROLL = 4


_GDN = lax.GatherDimensionNumbers(
    offset_dims=(), collapsed_slice_dims=(0,), start_index_map=(0,))


def _vgather(vec, idx):
    return lax.gather(vec, idx[:, None], _GDN, (1,),
                      mode=lax.GatherScatterMode.PROMISE_IN_BOUNDS)


def _sc_eval_vec(xv, tvec, pvec):
    v = xv * _SCALE
    xq = v.astype(jnp.int32)
    seg = jnp.zeros((16,), jnp.int32)
    for w in (8, 4, 2, 1):
        bpv = _vgather(tvec, seg + (w - 1))
        seg = seg + jnp.where(v >= bpv, w, 0)
    p = _vgather(pvec, seg)
    e = p & 7
    negm = (p << 28) >> 31
    inter = p >> 4
    sh = jnp.right_shift(xq, e)
    y = inter + ((sh ^ negm) - negm)
    return y.astype(jnp.float32) * (1.0 / _SCALE)


def _sc_plac(x, t16, pk16):
    mesh = plsc.VectorSubcoreMesh(core_axis_name="c", subcore_axis_name="s")

    @functools.partial(
        pl.kernel,
        mesh=mesh,
        out_type=jax.ShapeDtypeStruct((_N,), jnp.float32),
        scratch_types=[
            pltpu.VMEM((16,), jnp.float32),
            pltpu.VMEM((16,), jnp.int32),
            pltpu.VMEM((_CHUNK,), jnp.float32),
            pltpu.VMEM((_CHUNK,), jnp.float32),
        ],
    )
    def k(t_hbm, pk_hbm, x_hbm, o_hbm, tv, pkv, in0, out0):
        wid = lax.axis_index("s") * 2 + lax.axis_index("c")
        base = wid * _SHARD
        pltpu.sync_copy(t_hbm, tv)
        pltpu.sync_copy(pk_hbm, pkv)
        tvec = tv[...]
        pvec = pkv[...]

        def chunk_body(g, carry):
            off = base + g * _CHUNK
            pltpu.sync_copy(x_hbm.at[pl.ds(off, _CHUNK)], in0)

            def vec_body(j, c2):
                for u in range(_UNROLL):
                    s = (j * _UNROLL + u) * 16
                    out0[pl.ds(s, 16)] = _sc_eval_vec(in0[pl.ds(s, 16)], tvec, pvec)
                return c2

            lax.fori_loop(0, _VPC // _UNROLL, vec_body, 0)
            pltpu.sync_copy(out0, o_hbm.at[pl.ds(off, _CHUNK)])
            return carry

        lax.fori_loop(0, _NCHUNK, chunk_body, 0)

    return k(t16, pk16, x)


def _tables(breakpoints, intercepts, signs, exps):
    sneg = (signs < 0).astype(jnp.int32)
    packed = (intercepts << 4) | (sneg << 3) | exps
    t = jnp.where(breakpoints >= 1,
                  breakpoints.astype(jnp.float32),
                  jnp.float32(-0.99999994))
    return t, packed


def kernel(x, breakpoints, intercepts, signs, exps):
    t, packed = _tables(breakpoints, intercepts, signs, exps)
    t16 = jnp.concatenate([t, jnp.full((1,), jnp.inf, jnp.float32)])
    return _sc_plac(x, t16, packed)


# speedup vs baseline: 1.5978x; 1.5978x over previous
import functools

import jax
import jax.numpy as jnp
from jax import lax
from jax.experimental import pallas as pl
from jax.experimental.pallas import tpu as pltpu
from jax.experimental.pallas import tpu_sc as plsc

_SCALE = 65536.0
_N = 16777216

_NW = 32
_CHUNK = 16384
_SC_FRAC = 8
_N_SC = (_N // _NW) * _SC_FRAC
_N_TC = _N - _N_SC
_TC_BLK = 1048576


def _tables(breakpoints, intercepts, signs, exps):
    sneg = (signs < 0).astype(jnp.int32)
    packed = (intercepts << 4) | (sneg << 3) | exps
    t = jnp.where(breakpoints >= 1,
                  breakpoints.astype(jnp.float32),
                  jnp.float32(-0.99999994))
    return t, packed


def _tc_body(t_ref, pk_ref, x_ref, o_ref):
    v = x_ref[...] * _SCALE
    xq = v.astype(jnp.int32)
    T = [t_ref[i] for i in range(15)]
    P = [pk_ref[i] for i in range(16)]
    sel = jnp.where
    c1 = v >= T[7]
    c2 = v >= sel(c1, T[11], T[3])
    c3 = v >= sel(c2, sel(c1, T[13], T[5]), sel(c1, T[9], T[1]))
    c4 = v >= sel(c3,
                  sel(c2, sel(c1, T[14], T[6]), sel(c1, T[10], T[2])),
                  sel(c2, sel(c1, T[12], T[4]), sel(c1, T[8], T[0])))
    q = [sel(c4, P[2 * k + 1], P[2 * k]) for k in range(8)]
    r = [sel(c3, q[2 * k + 1], q[2 * k]) for k in range(4)]
    s = [sel(c2, r[2 * k + 1], r[2 * k]) for k in range(2)]
    p = sel(c1, s[1], s[0])
    e = p & 7
    negm = (p << 28) >> 31
    inter = p >> 4
    sh = jnp.right_shift(xq, e)
    y = inter + ((sh ^ negm) - negm)
    o_ref[...] = y.astype(jnp.float32) * (1.0 / _SCALE)


def _tc_plac(x_head, t, packed):
    n = x_head.shape[0]
    return pl.pallas_call(
        _tc_body,
        grid=(n // _TC_BLK,),
        in_specs=[
            pl.BlockSpec(memory_space=pltpu.SMEM),
            pl.BlockSpec(memory_space=pltpu.SMEM),
            pl.BlockSpec((_TC_BLK,), lambda i: (i,)),
        ],
        out_specs=pl.BlockSpec((_TC_BLK,), lambda i: (i,)),
        out_shape=jax.ShapeDtypeStruct((n,), jnp.float32),
    )(t, packed, x_head)


_GDN = lax.GatherDimensionNumbers(
    offset_dims=(), collapsed_slice_dims=(0,), start_index_map=(0,))


def _vgather(vec, idx):
    return lax.gather(vec, idx[:, None], _GDN, (1,),
                      mode=lax.GatherScatterMode.PROMISE_IN_BOUNDS)


def _sc_eval_vec(xv, tvec, pvec):
    v = xv * _SCALE
    xq = v.astype(jnp.int32)
    seg = jnp.zeros((16,), jnp.int32)
    for w in (8, 4, 2, 1):
        bpv = _vgather(tvec, seg + (w - 1))
        seg = seg + jnp.where(v >= bpv, w, 0)
    p = _vgather(pvec, seg)
    e = p & 7
    negm = (p << 28) >> 31
    inter = p >> 4
    sh = jnp.right_shift(xq, e)
    y = inter + ((sh ^ negm) - negm)
    return y.astype(jnp.float32) * (1.0 / _SCALE)


def _sc_plac(x_tail, t16, pk16):
    n = x_tail.shape[0]
    shard = n // _NW
    nchunk = shard // _CHUNK
    vpc = _CHUNK // 16
    unroll = 4
    mesh = plsc.VectorSubcoreMesh(core_axis_name="c", subcore_axis_name="s")

    @functools.partial(
        pl.kernel,
        mesh=mesh,
        out_type=jax.ShapeDtypeStruct((n,), jnp.float32),
        scratch_types=[
            pltpu.VMEM((16,), jnp.float32),
            pltpu.VMEM((16,), jnp.int32),
            pltpu.VMEM((_CHUNK,), jnp.float32),
            pltpu.VMEM((_CHUNK,), jnp.float32),
        ],
    )
    def k(t_hbm, pk_hbm, x_hbm, o_hbm, tv, pkv, in0, out0):
        wid = lax.axis_index("s") * 2 + lax.axis_index("c")
        base = wid * shard
        pltpu.sync_copy(t_hbm, tv)
        pltpu.sync_copy(pk_hbm, pkv)
        tvec = tv[...]
        pvec = pkv[...]

        def chunk_body(g, carry):
            off = base + g * _CHUNK
            pltpu.sync_copy(x_hbm.at[pl.ds(off, _CHUNK)], in0)

            def vec_body(j, c2):
                for u in range(unroll):
                    s = (j * unroll + u) * 16
                    out0[pl.ds(s, 16)] = _sc_eval_vec(in0[pl.ds(s, 16)], tvec, pvec)
                return c2

            lax.fori_loop(0, vpc // unroll, vec_body, 0)
            pltpu.sync_copy(out0, o_hbm.at[pl.ds(off, _CHUNK)])
            return carry

        lax.fori_loop(0, nchunk, chunk_body, 0)

    return k(t16, pk16, x_tail)


def kernel(x, breakpoints, intercepts, signs, exps):
    t, packed = _tables(breakpoints, intercepts, signs, exps)
    t16 = jnp.concatenate([t, jnp.full((1,), jnp.inf, jnp.float32)])
    y_sc = _sc_plac(x[_N_TC:], t16, packed)
    y_tc = _tc_plac(x[:_N_TC], t, packed)
    return jnp.concatenate([y_tc, y_sc])
